# Initial kernel scaffold; baseline (speedup 1.0000x reference)
import jax
import jax.numpy as jnp
from jax.experimental import pallas as pl


def kernel(x, edge_index, W_e1, b_e1, g_e1, be_e1, W_e2, b_e2, g_e2, be_e2, W1, W2, W3, W_s, b_s, g_s, be_s, W_p, b_p, g_p, be_p, W_d, b_d, g_d, be_d):
    raise NotImplementedError("write your pallas kernel here")


# speedup vs baseline: 13.0120x; 13.0120x over previous
import functools

import jax
import jax.numpy as jnp
import numpy as np
from jax import lax
from jax.experimental import pallas as pl
from jax.experimental.pallas import tpu as pltpu
from jax.experimental.pallas import tpu_sc as plsc

N = 10000
E = 320000
D_IN = 128
FH1 = 64
FH2 = 32
GH1 = 32
GH2 = 16
SH = 16
LAT = FH2 + GH2
F = 32
_BN = 1.0 / float(np.sqrt(1.001))

NC = 2
NS = 16
NW = NC * NS
EPT = E // NW
CH = 80
NCH = EPT // CH
RPT = 624
RTAIL = N - NS * RPT

BR = 1000
GRID = N // BR


_sc_mesh = plsc.VectorSubcoreMesh(
    core_axis_name="c", subcore_axis_name="s", num_cores=NC, num_subcores=NS
)


@functools.partial(
    pl.kernel,
    out_type=jax.ShapeDtypeStruct((NC * N, F), jnp.float32),
    mesh=_sc_mesh,
    compiler_params=pltpu.CompilerParams(use_tc_tiling_on_sc=False),
    scratch_types=[
        pltpu.VMEM((NCH, CH), jnp.int32),
        pltpu.VMEM((NCH, CH), jnp.int32),
        pltpu.VMEM((CH, F), jnp.float32),
        pltpu.VMEM_SHARED((N, F), jnp.float32),
        pltpu.SemaphoreType.DMA,
    ],
)
def _spmm_sc(table, rowi, coli, zeros, out, colbuf, rowbuf, rows, acc, sem):
    cid = lax.axis_index("c")
    sid = lax.axis_index("s")
    wid = sid * NC + cid

    pltpu.sync_copy(coli.at[wid], colbuf)
    pltpu.sync_copy(rowi.at[wid], rowbuf)

    pltpu.sync_copy(zeros.at[pl.ds(sid * RPT, RPT)], acc.at[pl.ds(sid * RPT, RPT)])

    @pl.when(sid == NS - 1)
    def _zero_tail():
        pltpu.sync_copy(zeros.at[pl.ds(NS * RPT, RTAIL)], acc.at[pl.ds(NS * RPT, RTAIL)])

    plsc.subcore_barrier()

    def chunk(i, carry):
        pltpu.async_copy(table.at[colbuf.at[i]], rows, sem).wait()
        pltpu.sync_copy(rows, acc.at[rowbuf.at[i]], add=True)
        return carry

    lax.fori_loop(0, NCH, chunk, 0)

    plsc.subcore_barrier()
    pltpu.sync_copy(
        acc.at[pl.ds(sid * RPT, RPT)],
        out.at[pl.ds(cid * N + sid * RPT, RPT)],
    )

    @pl.when(sid == NS - 1)
    def _write_tail():
        pltpu.sync_copy(
            acc.at[pl.ds(NS * RPT, RTAIL)],
            out.at[pl.ds(cid * N + NS * RPT, RTAIL)],
        )


def _elu(h):
    return jnp.where(h > 0.0, h, jnp.exp(jnp.minimum(h, 0.0)) - 1.0)


def _enc_body(x_ref, w1_ref, s1_ref, t1_ref, w2_ref, s2_ref, t2_ref, o_ref):
    h = jnp.dot(x_ref[...], w1_ref[...], preferred_element_type=jnp.float32)
    h = _elu(h * s1_ref[...] + t1_ref[...])
    h = jnp.dot(h, w2_ref[...], preferred_element_type=jnp.float32)
    o_ref[...] = _elu(h * s2_ref[...] + t2_ref[...])


def _hidden_body(p0_ref, p1_ref, w1_ref, o_ref):
    agg = p0_ref[...] + p1_ref[...]
    h = jnp.dot(agg, w1_ref[...], preferred_element_type=jnp.float32)
    o_ref[...] = jnp.maximum(h, 0.0)


def _head_body(feat_ref, q0_ref, q1_ref, w2_ref, w3_ref,
               ws_ref, ss_ref, ts_ref, wp_ref, sp_ref, tp_ref,
               wd_ref, sd_ref, td_ref,
               mu_ref, ls_ref, zs_ref, zp_ref, dec_ref):
    agg = q0_ref[...] + q1_ref[...]
    mu = jnp.dot(agg, w2_ref[...], preferred_element_type=jnp.float32)
    mu_ref[...] = mu
    ls_ref[...] = jnp.dot(agg, w3_ref[...], preferred_element_type=jnp.float32)
    z = jnp.concatenate([feat_ref[...], mu], axis=1)
    zs = _elu(jnp.dot(z, ws_ref[...], preferred_element_type=jnp.float32)
              * ss_ref[...] + ts_ref[...])
    zp = _elu(jnp.dot(z, wp_ref[...], preferred_element_type=jnp.float32)
              * sp_ref[...] + tp_ref[...])
    zs_ref[...] = zs
    zp_ref[...] = zp
    comb = jnp.concatenate([zs, zp], axis=1)
    dec_ref[...] = _elu(jnp.dot(comb, wd_ref[...], preferred_element_type=jnp.float32)
                        * sd_ref[...] + td_ref[...])


def _row_spec(width):
    return pl.BlockSpec((BR, width), lambda i: (i, 0))


def _full_spec(shape):
    return pl.BlockSpec(shape, lambda i: (0,) * len(shape))


def _scale_shift(b, g, be):
    s = (g * _BN).reshape(1, -1)
    t = (be + b * g * _BN).reshape(1, -1)
    return s, t


def kernel(x, edge_index, W_e1, b_e1, g_e1, be_e1, W_e2, b_e2, g_e2, be_e2,
           W1, W2, W3, W_s, b_s, g_s, be_s, W_p, b_p, g_p, be_p,
           W_d, b_d, g_d, be_d):
    s1, t1 = _scale_shift(b_e1, g_e1, be_e1)
    s2, t2 = _scale_shift(b_e2, g_e2, be_e2)
    ss, ts = _scale_shift(b_s, g_s, be_s)
    sp, tp = _scale_shift(b_p, g_p, be_p)
    sd, td = _scale_shift(b_d, g_d, be_d)

    rowi = edge_index[0].reshape(NW, NCH, CH)
    coli = edge_index[1].reshape(NW, NCH, CH)
    zeros = jnp.zeros((N, F), jnp.float32)

    feat_x = pl.pallas_call(
        _enc_body,
        grid=(GRID,),
        in_specs=[
            _row_spec(D_IN),
            _full_spec((D_IN, FH1)), _full_spec((1, FH1)), _full_spec((1, FH1)),
            _full_spec((FH1, FH2)), _full_spec((1, FH2)), _full_spec((1, FH2)),
        ],
        out_specs=_row_spec(FH2),
        out_shape=jax.ShapeDtypeStruct((N, FH2), jnp.float32),
    )(x, W_e1, s1, t1, W_e2, s2, t2)

    part1 = _spmm_sc(feat_x, rowi, coli, zeros)
    p0, p1 = part1[:N], part1[N:]

    hidden1 = pl.pallas_call(
        _hidden_body,
        grid=(GRID,),
        in_specs=[_row_spec(F), _row_spec(F), _full_spec((FH2, GH1))],
        out_specs=_row_spec(GH1),
        out_shape=jax.ShapeDtypeStruct((N, GH1), jnp.float32),
    )(p0, p1, W1)

    part2 = _spmm_sc(hidden1, rowi, coli, zeros)
    q0, q1 = part2[:N], part2[N:]

    mu, ls, zs, zp, dec = pl.pallas_call(
        _head_body,
        grid=(GRID,),
        in_specs=[
            _row_spec(FH2), _row_spec(F), _row_spec(F),
            _full_spec((GH1, GH2)), _full_spec((GH1, GH2)),
            _full_spec((LAT, SH)), _full_spec((1, SH)), _full_spec((1, SH)),
            _full_spec((LAT, LAT - SH)), _full_spec((1, LAT - SH)), _full_spec((1, LAT - SH)),
            _full_spec((LAT, D_IN)), _full_spec((1, D_IN)), _full_spec((1, D_IN)),
        ],
        out_specs=[
            _row_spec(GH2), _row_spec(GH2), _row_spec(SH),
            _row_spec(LAT - SH), _row_spec(D_IN),
        ],
        out_shape=[
            jax.ShapeDtypeStruct((N, GH2), jnp.float32),
            jax.ShapeDtypeStruct((N, GH2), jnp.float32),
            jax.ShapeDtypeStruct((N, SH), jnp.float32),
            jax.ShapeDtypeStruct((N, LAT - SH), jnp.float32),
            jax.ShapeDtypeStruct((N, D_IN), jnp.float32),
        ],
    )(feat_x, q0, q1, W2, W3, W_s, ss, ts, W_p, sp, tp, W_d, sd, td)

    return (mu, ls, zs, zp, dec)


# speedup vs baseline: 14.2221x; 1.0930x over previous
import functools

import jax
import jax.numpy as jnp
import numpy as np
from jax import lax
from jax.experimental import pallas as pl
from jax.experimental.pallas import tpu as pltpu
from jax.experimental.pallas import tpu_sc as plsc

N = 10000
E = 320000
D_IN = 128
FH1 = 64
FH2 = 32
GH1 = 32
GH2 = 16
SH = 16
LAT = FH2 + GH2
F = 32
_BN = 1.0 / float(np.sqrt(1.001))

NC = 2
NS = 16
NW = NC * NS
EPT = E // NW
CH = 128
NCH = 80
EPAD = NCH * CH - EPT
NACC = N + 8
NBUF = 8
PF = 4
RPT = 624
RTAIL = N - NS * RPT

BR = 1000
GRID = N // BR


_sc_mesh = plsc.VectorSubcoreMesh(
    core_axis_name="c", subcore_axis_name="s", num_cores=NC, num_subcores=NS
)


@functools.partial(
    pl.kernel,
    out_type=jax.ShapeDtypeStruct((NC * N, F), jnp.float32),
    mesh=_sc_mesh,
    compiler_params=pltpu.CompilerParams(use_tc_tiling_on_sc=False),
    scratch_types=[
        pltpu.VMEM((NCH, CH), jnp.int32),
        pltpu.VMEM((NCH, CH), jnp.int32),
        pltpu.VMEM((NBUF, CH, F), jnp.float32),
        pltpu.VMEM_SHARED((NACC, F), jnp.float32),
    ]
    + [pltpu.SemaphoreType.DMA] * (2 * NBUF),
)
def _spmm_sc(table, rowi, coli, zeros, out, colbuf, rowbuf, rows, acc, *sems):
    gsems = sems[:NBUF]
    ssems = sems[NBUF:]
    cid = lax.axis_index("c")
    sid = lax.axis_index("s")
    wid = sid * NC + cid

    pltpu.sync_copy(coli.at[wid], colbuf)
    pltpu.sync_copy(rowi.at[wid], rowbuf)

    pltpu.sync_copy(zeros.at[pl.ds(sid * RPT, RPT)], acc.at[pl.ds(sid * RPT, RPT)])

    @pl.when(sid == NS - 1)
    def _zero_tail():
        pltpu.sync_copy(
            zeros.at[pl.ds(NS * RPT, NACC - NS * RPT)],
            acc.at[pl.ds(NS * RPT, NACC - NS * RPT)],
        )

    plsc.subcore_barrier()

    def _gather(i, slot):
        return pltpu.async_copy(table.at[colbuf.at[i]], rows.at[slot], gsems[slot])

    def _gather_wait(i, slot):
        pltpu.make_async_copy(table.at[colbuf.at[i]], rows.at[slot], gsems[slot]).wait()

    def _scatter(i, slot):
        return pltpu.async_copy(
            rows.at[slot], acc.at[rowbuf.at[i]], ssems[slot], add=True
        )

    def _scatter_wait(i, slot):
        pltpu.make_async_copy(rows.at[slot], acc.at[rowbuf.at[i]], ssems[slot]).wait()

    for b in range(PF):
        _gather(b, b)

    @pl.loop(0, NCH // NBUF)
    def _outer(r):
        base = r * NBUF
        for b in range(NBUF):
            i = base + b
            _gather_wait(i, b)
            _scatter(i, b)
            j = i + PF
            jslot = (b + PF) % NBUF

            @pl.when(j >= NBUF)
            def _drain():
                _scatter_wait(j - NBUF, jslot)

            @pl.when(j < NCH)
            def _prefetch():
                _gather(j, jslot)

    for b in range(PF):
        i = NCH - PF + b
        _scatter_wait(i, (i % NBUF))

    plsc.subcore_barrier()
    pltpu.sync_copy(
        acc.at[pl.ds(sid * RPT, RPT)],
        out.at[pl.ds(cid * N + sid * RPT, RPT)],
    )

    @pl.when(sid == NS - 1)
    def _write_tail():
        pltpu.sync_copy(
            acc.at[pl.ds(NS * RPT, RTAIL)],
            out.at[pl.ds(cid * N + NS * RPT, RTAIL)],
        )


def _elu(h):
    return jnp.where(h > 0.0, h, jnp.exp(jnp.minimum(h, 0.0)) - 1.0)


def _enc_body(x_ref, w1_ref, s1_ref, t1_ref, w2_ref, s2_ref, t2_ref, o_ref):
    h = jnp.dot(x_ref[...], w1_ref[...], preferred_element_type=jnp.float32)
    h = _elu(h * s1_ref[...] + t1_ref[...])
    h = jnp.dot(h, w2_ref[...], preferred_element_type=jnp.float32)
    o_ref[...] = _elu(h * s2_ref[...] + t2_ref[...])


def _hidden_body(p0_ref, p1_ref, w1_ref, o_ref):
    agg = p0_ref[...] + p1_ref[...]
    h = jnp.dot(agg, w1_ref[...], preferred_element_type=jnp.float32)
    o_ref[...] = jnp.maximum(h, 0.0)


def _head_body(feat_ref, q0_ref, q1_ref, w2_ref, w3_ref,
               ws_ref, ss_ref, ts_ref, wp_ref, sp_ref, tp_ref,
               wd_ref, sd_ref, td_ref,
               mu_ref, ls_ref, zs_ref, zp_ref, dec_ref):
    agg = q0_ref[...] + q1_ref[...]
    mu = jnp.dot(agg, w2_ref[...], preferred_element_type=jnp.float32)
    mu_ref[...] = mu
    ls_ref[...] = jnp.dot(agg, w3_ref[...], preferred_element_type=jnp.float32)
    z = jnp.concatenate([feat_ref[...], mu], axis=1)
    zs = _elu(jnp.dot(z, ws_ref[...], preferred_element_type=jnp.float32)
              * ss_ref[...] + ts_ref[...])
    zp = _elu(jnp.dot(z, wp_ref[...], preferred_element_type=jnp.float32)
              * sp_ref[...] + tp_ref[...])
    zs_ref[...] = zs
    zp_ref[...] = zp
    comb = jnp.concatenate([zs, zp], axis=1)
    dec_ref[...] = _elu(jnp.dot(comb, wd_ref[...], preferred_element_type=jnp.float32)
                        * sd_ref[...] + td_ref[...])


def _row_spec(width):
    return pl.BlockSpec((BR, width), lambda i: (i, 0))


def _p0_spec(width):
    return pl.BlockSpec((BR, width), lambda i: (i, 0))


def _p1_spec(width):
    return pl.BlockSpec((BR, width), lambda i: (i + GRID, 0))


def _full_spec(shape):
    return pl.BlockSpec(shape, lambda i: (0,) * len(shape))


def _scale_shift(b, g, be):
    s = (g * _BN).reshape(1, -1)
    t = (be + b * g * _BN).reshape(1, -1)
    return s, t


def kernel(x, edge_index, W_e1, b_e1, g_e1, be_e1, W_e2, b_e2, g_e2, be_e2,
           W1, W2, W3, W_s, b_s, g_s, be_s, W_p, b_p, g_p, be_p,
           W_d, b_d, g_d, be_d):
    s1, t1 = _scale_shift(b_e1, g_e1, be_e1)
    s2, t2 = _scale_shift(b_e2, g_e2, be_e2)
    ss, ts = _scale_shift(b_s, g_s, be_s)
    sp, tp = _scale_shift(b_p, g_p, be_p)
    sd, td = _scale_shift(b_d, g_d, be_d)

    rowi = jnp.pad(edge_index[0].reshape(NW, EPT), ((0, 0), (0, EPAD)),
                   constant_values=N).reshape(NW, NCH, CH)
    coli = jnp.pad(edge_index[1].reshape(NW, EPT), ((0, 0), (0, EPAD)),
                   constant_values=0).reshape(NW, NCH, CH)
    zeros = jnp.zeros((NACC, F), jnp.float32)

    feat_x = pl.pallas_call(
        _enc_body,
        grid=(GRID,),
        in_specs=[
            _row_spec(D_IN),
            _full_spec((D_IN, FH1)), _full_spec((1, FH1)), _full_spec((1, FH1)),
            _full_spec((FH1, FH2)), _full_spec((1, FH2)), _full_spec((1, FH2)),
        ],
        out_specs=_row_spec(FH2),
        out_shape=jax.ShapeDtypeStruct((N, FH2), jnp.float32),
    )(x, W_e1, s1, t1, W_e2, s2, t2)

    part1 = _spmm_sc(feat_x, rowi, coli, zeros)

    hidden1 = pl.pallas_call(
        _hidden_body,
        grid=(GRID,),
        in_specs=[_p0_spec(F), _p1_spec(F), _full_spec((FH2, GH1))],
        out_specs=_row_spec(GH1),
        out_shape=jax.ShapeDtypeStruct((N, GH1), jnp.float32),
    )(part1, part1, W1)

    part2 = _spmm_sc(hidden1, rowi, coli, zeros)

    mu, ls, zs, zp, dec = pl.pallas_call(
        _head_body,
        grid=(GRID,),
        in_specs=[
            _row_spec(FH2), _p0_spec(F), _p1_spec(F),
            _full_spec((GH1, GH2)), _full_spec((GH1, GH2)),
            _full_spec((LAT, SH)), _full_spec((1, SH)), _full_spec((1, SH)),
            _full_spec((LAT, LAT - SH)), _full_spec((1, LAT - SH)), _full_spec((1, LAT - SH)),
            _full_spec((LAT, D_IN)), _full_spec((1, D_IN)), _full_spec((1, D_IN)),
        ],
        out_specs=[
            _row_spec(GH2), _row_spec(GH2), _row_spec(SH),
            _row_spec(LAT - SH), _row_spec(D_IN),
        ],
        out_shape=[
            jax.ShapeDtypeStruct((N, GH2), jnp.float32),
            jax.ShapeDtypeStruct((N, GH2), jnp.float32),
            jax.ShapeDtypeStruct((N, SH), jnp.float32),
            jax.ShapeDtypeStruct((N, LAT - SH), jnp.float32),
            jax.ShapeDtypeStruct((N, D_IN), jnp.float32),
        ],
    )(feat_x, part2, part2, W2, W3, W_s, ss, ts, W_p, sp, tp, W_d, sd, td)

    return (mu, ls, zs, zp, dec)


# speedup vs baseline: 22.8920x; 1.6096x over previous
import functools

import jax
import jax.numpy as jnp
import numpy as np
from jax import lax
from jax.experimental import pallas as pl
from jax.experimental.pallas import tpu as pltpu
from jax.experimental.pallas import tpu_sc as plsc

N = 10000
E = 320000
D_IN = 128
FH1 = 64
FH2 = 32
GH1 = 32
GH2 = 16
SH = 16
LAT = FH2 + GH2
F = 32
_BN = 1.0 / float(np.sqrt(1.001))

NC = 2
NS = 16
NW = NC * NS
EPT = E // NW
CH = 128
NCH = EPT // CH
ETAIL = EPT - NCH * CH
NBUF = 6
PF = 4
RPT = 624
RTAIL = N - NS * RPT

BR = 2000
GRID = N // BR


_sc_mesh = plsc.VectorSubcoreMesh(
    core_axis_name="c", subcore_axis_name="s", num_cores=NC, num_subcores=NS
)


@functools.partial(
    pl.kernel,
    out_type=jax.ShapeDtypeStruct((NC * N, F), jnp.float32),
    mesh=_sc_mesh,
    compiler_params=pltpu.CompilerParams(use_tc_tiling_on_sc=False),
    scratch_types=[
        pltpu.VMEM((EPT,), jnp.int32),
        pltpu.VMEM((NBUF, CH, F), jnp.float32),
        pltpu.VMEM_SHARED((N, F), jnp.float32),
        pltpu.VMEM_SHARED((N, F), jnp.float32),
        pltpu.VMEM((ETAIL,), jnp.int32),
    ]
    + [pltpu.VMEM((CH,), jnp.int32)] * NBUF
    + [pltpu.SemaphoreType.DMA] * (2 * NBUF),
)
def _spmm_sc(table, rowi, coli, zeros, out, colbuf, rows, acc,
             sp_table, rtail, *rest):
    rbuf = rest[:NBUF]
    gsems = rest[NBUF:2 * NBUF]
    ssems = rest[2 * NBUF:]
    cid = lax.axis_index("c")
    sid = lax.axis_index("s")
    wid = sid * NC + cid

    ebase = wid * EPT
    pltpu.sync_copy(coli.at[pl.ds(ebase, EPT)], colbuf)

    pltpu.sync_copy(zeros.at[pl.ds(sid * RPT, RPT)], acc.at[pl.ds(sid * RPT, RPT)])
    pltpu.sync_copy(table.at[pl.ds(sid * RPT, RPT)], sp_table.at[pl.ds(sid * RPT, RPT)])

    @pl.when(sid == NS - 1)
    def _stage_tail():
        pltpu.sync_copy(
            zeros.at[pl.ds(NS * RPT, RTAIL)],
            acc.at[pl.ds(NS * RPT, RTAIL)],
        )
        pltpu.sync_copy(
            table.at[pl.ds(NS * RPT, RTAIL)],
            sp_table.at[pl.ds(NS * RPT, RTAIL)],
        )

    plsc.subcore_barrier()

    def _gather(i, slot):
        pltpu.sync_copy(rowi.at[pl.ds(ebase + i * CH, CH)], rbuf[slot])
        return pltpu.async_copy(
            sp_table.at[colbuf.at[pl.ds(i * CH, CH)]], rows.at[slot], gsems[slot]
        )

    def _gather_wait(i, slot):
        pltpu.make_async_copy(
            sp_table.at[colbuf.at[pl.ds(i * CH, CH)]], rows.at[slot], gsems[slot]
        ).wait()

    def _scatter(slot):
        return pltpu.async_copy(rows.at[slot], acc.at[rbuf[slot]], ssems[slot], add=True)

    def _scatter_wait(slot):
        pltpu.make_async_copy(rows.at[slot], acc.at[rbuf[slot]], ssems[slot]).wait()

    for b in range(PF):
        _gather(b, b)

    @pl.loop(0, NCH // NBUF)
    def _outer(r):
        base = r * NBUF
        for b in range(NBUF):
            i = base + b
            _gather_wait(i, b)
            _scatter(b)
            j = i + PF
            jslot = (b + PF) % NBUF

            @pl.when(j >= NBUF)
            def _drain():
                _scatter_wait(jslot)

            @pl.when(j < NCH)
            def _prefetch():
                _gather(j, jslot)

    for b in range(NBUF - PF):
        _scatter_wait((NCH - (NBUF - PF) + b) % NBUF)

    pltpu.sync_copy(rowi.at[pl.ds(ebase + NCH * CH, ETAIL)], rtail)
    pltpu.async_copy(
        sp_table.at[colbuf.at[pl.ds(NCH * CH, ETAIL)]],
        rows.at[0, pl.ds(0, ETAIL)],
        gsems[0],
    ).wait()
    pltpu.sync_copy(rows.at[0, pl.ds(0, ETAIL)], acc.at[rtail], add=True)

    plsc.subcore_barrier()
    pltpu.sync_copy(
        acc.at[pl.ds(sid * RPT, RPT)],
        out.at[pl.ds(cid * N + sid * RPT, RPT)],
    )

    @pl.when(sid == NS - 1)
    def _write_tail():
        pltpu.sync_copy(
            acc.at[pl.ds(NS * RPT, RTAIL)],
            out.at[pl.ds(cid * N + NS * RPT, RTAIL)],
        )


def _elu(h):
    return jnp.where(h > 0.0, h, jnp.exp(jnp.minimum(h, 0.0)) - 1.0)


def _enc_body(x_ref, w1_ref, s1_ref, t1_ref, w2_ref, s2_ref, t2_ref, o_ref):
    h = jnp.dot(x_ref[...], w1_ref[...], preferred_element_type=jnp.float32)
    h = _elu(h * s1_ref[...] + t1_ref[...])
    h = jnp.dot(h, w2_ref[...], preferred_element_type=jnp.float32)
    o_ref[...] = _elu(h * s2_ref[...] + t2_ref[...])


def _hidden_body(p0_ref, p1_ref, w1_ref, o_ref):
    agg = p0_ref[...] + p1_ref[...]
    h = jnp.dot(agg, w1_ref[...], preferred_element_type=jnp.float32)
    o_ref[...] = jnp.maximum(h, 0.0)


def _head_body(feat_ref, q0_ref, q1_ref, w2_ref, w3_ref,
               ws_ref, ss_ref, ts_ref, wp_ref, sp_ref, tp_ref,
               wd_ref, sd_ref, td_ref,
               mu_ref, ls_ref, zs_ref, zp_ref, dec_ref):
    agg = q0_ref[...] + q1_ref[...]
    mu = jnp.dot(agg, w2_ref[...], preferred_element_type=jnp.float32)
    mu_ref[...] = mu
    ls_ref[...] = jnp.dot(agg, w3_ref[...], preferred_element_type=jnp.float32)
    z = jnp.concatenate([feat_ref[...], mu], axis=1)
    zs = _elu(jnp.dot(z, ws_ref[...], preferred_element_type=jnp.float32)
              * ss_ref[...] + ts_ref[...])
    zp = _elu(jnp.dot(z, wp_ref[...], preferred_element_type=jnp.float32)
              * sp_ref[...] + tp_ref[...])
    zs_ref[...] = zs
    zp_ref[...] = zp
    comb = jnp.concatenate([zs, zp], axis=1)
    dec_ref[...] = _elu(jnp.dot(comb, wd_ref[...], preferred_element_type=jnp.float32)
                        * sd_ref[...] + td_ref[...])


def _row_spec(width):
    return pl.BlockSpec((BR, width), lambda i: (i, 0))


def _p0_spec(width):
    return pl.BlockSpec((BR, width), lambda i: (i, 0))


def _p1_spec(width):
    return pl.BlockSpec((BR, width), lambda i: (i + GRID, 0))


def _full_spec(shape):
    return pl.BlockSpec(shape, lambda i: (0,) * len(shape))


def _scale_shift(b, g, be):
    s = (g * _BN).reshape(1, -1)
    t = (be + b * g * _BN).reshape(1, -1)
    return s, t


def kernel(x, edge_index, W_e1, b_e1, g_e1, be_e1, W_e2, b_e2, g_e2, be_e2,
           W1, W2, W3, W_s, b_s, g_s, be_s, W_p, b_p, g_p, be_p,
           W_d, b_d, g_d, be_d):
    s1, t1 = _scale_shift(b_e1, g_e1, be_e1)
    s2, t2 = _scale_shift(b_e2, g_e2, be_e2)
    ss, ts = _scale_shift(b_s, g_s, be_s)
    sp, tp = _scale_shift(b_p, g_p, be_p)
    sd, td = _scale_shift(b_d, g_d, be_d)

    rowi = edge_index[0]
    coli = edge_index[1]
    zeros = jnp.zeros((N, F), jnp.float32)

    feat_x = pl.pallas_call(
        _enc_body,
        grid=(GRID,),
        in_specs=[
            _row_spec(D_IN),
            _full_spec((D_IN, FH1)), _full_spec((1, FH1)), _full_spec((1, FH1)),
            _full_spec((FH1, FH2)), _full_spec((1, FH2)), _full_spec((1, FH2)),
        ],
        out_specs=_row_spec(FH2),
        out_shape=jax.ShapeDtypeStruct((N, FH2), jnp.float32),
    )(x, W_e1, s1, t1, W_e2, s2, t2)

    part1 = _spmm_sc(feat_x, rowi, coli, zeros)

    hidden1 = pl.pallas_call(
        _hidden_body,
        grid=(GRID,),
        in_specs=[_p0_spec(F), _p1_spec(F), _full_spec((FH2, GH1))],
        out_specs=_row_spec(GH1),
        out_shape=jax.ShapeDtypeStruct((N, GH1), jnp.float32),
    )(part1, part1, W1)

    part2 = _spmm_sc(hidden1, rowi, coli, zeros)

    mu, ls, zs, zp, dec = pl.pallas_call(
        _head_body,
        grid=(GRID,),
        in_specs=[
            _row_spec(FH2), _p0_spec(F), _p1_spec(F),
            _full_spec((GH1, GH2)), _full_spec((GH1, GH2)),
            _full_spec((LAT, SH)), _full_spec((1, SH)), _full_spec((1, SH)),
            _full_spec((LAT, LAT - SH)), _full_spec((1, LAT - SH)), _full_spec((1, LAT - SH)),
            _full_spec((LAT, D_IN)), _full_spec((1, D_IN)), _full_spec((1, D_IN)),
        ],
        out_specs=[
            _row_spec(GH2), _row_spec(GH2), _row_spec(SH),
            _row_spec(LAT - SH), _row_spec(D_IN),
        ],
        out_shape=[
            jax.ShapeDtypeStruct((N, GH2), jnp.float32),
            jax.ShapeDtypeStruct((N, GH2), jnp.float32),
            jax.ShapeDtypeStruct((N, SH), jnp.float32),
            jax.ShapeDtypeStruct((N, LAT - SH), jnp.float32),
            jax.ShapeDtypeStruct((N, D_IN), jnp.float32),
        ],
    )(feat_x, part2, part2, W2, W3, W_s, ss, ts, W_p, sp, tp, W_d, sd, td)

    return (mu, ls, zs, zp, dec)


# speedup vs baseline: 25.9885x; 1.1353x over previous
import functools

import jax
import jax.numpy as jnp
import numpy as np
from jax import lax
from jax.experimental import pallas as pl
from jax.experimental.pallas import tpu as pltpu
from jax.experimental.pallas import tpu_sc as plsc

N = 10000
E = 320000
D_IN = 128
FH1 = 64
FH2 = 32
GH1 = 32
GH2 = 16
SH = 16
LAT = FH2 + GH2
F = 32
_BN = 1.0 / float(np.sqrt(1.001))

NC = 2
NS = 16
NW = NC * NS
EPT = E // NW
CH = 128
NCH = EPT // CH
ETAIL = EPT - NCH * CH
NBUF = 6
PF = 4
RPT = 624
RTAIL = N - NS * RPT

BR = 2000
GRID = N // BR


_sc_mesh = plsc.VectorSubcoreMesh(
    core_axis_name="c", subcore_axis_name="s", num_cores=NC, num_subcores=NS
)


@functools.partial(
    pl.kernel,
    out_type=jax.ShapeDtypeStruct((NC * N, F), jnp.float32),
    mesh=_sc_mesh,
    compiler_params=pltpu.CompilerParams(use_tc_tiling_on_sc=False),
    scratch_types=[
        pltpu.VMEM((EPT,), jnp.int32),
        pltpu.VMEM((NBUF, CH, F), jnp.float32),
        pltpu.VMEM_SHARED((N, F), jnp.float32),
        pltpu.VMEM_SHARED((N, F), jnp.float32),
        pltpu.VMEM((ETAIL,), jnp.int32),
    ]
    + [pltpu.VMEM((CH,), jnp.int32)] * NBUF
    + [pltpu.SemaphoreType.DMA] * (3 * NBUF),
)
def _spmm_sc(table, rowi, coli, zeros, out, colbuf, rows, acc,
             sp_table, rtail, *rest):
    rbuf = rest[:NBUF]
    gsems = rest[NBUF:2 * NBUF]
    ssems = rest[2 * NBUF:3 * NBUF]
    rsems = rest[3 * NBUF:]
    cid = lax.axis_index("c")
    sid = lax.axis_index("s")
    wid = sid * NC + cid

    ebase = wid * EPT
    pltpu.sync_copy(coli.at[pl.ds(ebase, EPT)], colbuf)

    pltpu.sync_copy(zeros.at[pl.ds(sid * RPT, RPT)], acc.at[pl.ds(sid * RPT, RPT)])
    pltpu.sync_copy(table.at[pl.ds(sid * RPT, RPT)], sp_table.at[pl.ds(sid * RPT, RPT)])

    @pl.when(sid == NS - 1)
    def _stage_tail():
        pltpu.sync_copy(
            zeros.at[pl.ds(NS * RPT, RTAIL)],
            acc.at[pl.ds(NS * RPT, RTAIL)],
        )
        pltpu.sync_copy(
            table.at[pl.ds(NS * RPT, RTAIL)],
            sp_table.at[pl.ds(NS * RPT, RTAIL)],
        )

    plsc.subcore_barrier()

    def _gather(i, slot):
        pltpu.async_copy(rowi.at[pl.ds(ebase + i * CH, CH)], rbuf[slot], rsems[slot])
        return pltpu.async_copy(
            sp_table.at[colbuf.at[pl.ds(i * CH, CH)]], rows.at[slot], gsems[slot]
        )

    def _gather_wait(i, slot):
        pltpu.make_async_copy(
            sp_table.at[colbuf.at[pl.ds(i * CH, CH)]], rows.at[slot], gsems[slot]
        ).wait()

    def _scatter(i, slot):
        pltpu.make_async_copy(
            rowi.at[pl.ds(ebase + i * CH, CH)], rbuf[slot], rsems[slot]
        ).wait()
        return pltpu.async_copy(rows.at[slot], acc.at[rbuf[slot]], ssems[slot], add=True)

    def _scatter_wait(slot):
        pltpu.make_async_copy(rows.at[slot], acc.at[rbuf[slot]], ssems[slot]).wait()

    for b in range(PF):
        _gather(b, b)

    @pl.loop(0, NCH // NBUF)
    def _outer(r):
        base = r * NBUF
        for b in range(NBUF):
            i = base + b
            _gather_wait(i, b)
            _scatter(i, b)
            j = i + PF
            jslot = (b + PF) % NBUF

            @pl.when(j >= NBUF)
            def _drain():
                _scatter_wait(jslot)

            @pl.when(j < NCH)
            def _prefetch():
                _gather(j, jslot)

    for b in range(NBUF - PF):
        _scatter_wait((NCH - (NBUF - PF) + b) % NBUF)

    pltpu.sync_copy(rowi.at[pl.ds(ebase + NCH * CH, ETAIL)], rtail)
    pltpu.async_copy(
        sp_table.at[colbuf.at[pl.ds(NCH * CH, ETAIL)]],
        rows.at[0, pl.ds(0, ETAIL)],
        gsems[0],
    ).wait()
    pltpu.sync_copy(rows.at[0, pl.ds(0, ETAIL)], acc.at[rtail], add=True)

    plsc.subcore_barrier()
    pltpu.sync_copy(
        acc.at[pl.ds(sid * RPT, RPT)],
        out.at[pl.ds(cid * N + sid * RPT, RPT)],
    )

    @pl.when(sid == NS - 1)
    def _write_tail():
        pltpu.sync_copy(
            acc.at[pl.ds(NS * RPT, RTAIL)],
            out.at[pl.ds(cid * N + NS * RPT, RTAIL)],
        )


def _elu(h):
    return jnp.where(h > 0.0, h, jnp.exp(jnp.minimum(h, 0.0)) - 1.0)


def _enc_body(x_ref, w1_ref, s1_ref, t1_ref, w2_ref, s2_ref, t2_ref, o_ref):
    h = jnp.dot(x_ref[...], w1_ref[...], preferred_element_type=jnp.float32)
    h = _elu(h * s1_ref[...] + t1_ref[...])
    h = jnp.dot(h, w2_ref[...], preferred_element_type=jnp.float32)
    o_ref[...] = _elu(h * s2_ref[...] + t2_ref[...])


def _hidden_body(p0_ref, p1_ref, w1_ref, o_ref):
    agg = p0_ref[...] + p1_ref[...]
    h = jnp.dot(agg, w1_ref[...], preferred_element_type=jnp.float32)
    o_ref[...] = jnp.maximum(h, 0.0)


def _head_body(feat_ref, q0_ref, q1_ref, w2_ref, w3_ref,
               ws_ref, ss_ref, ts_ref, wp_ref, sp_ref, tp_ref,
               wd_ref, sd_ref, td_ref,
               mu_ref, ls_ref, zs_ref, zp_ref, dec_ref):
    agg = q0_ref[...] + q1_ref[...]
    mu = jnp.dot(agg, w2_ref[...], preferred_element_type=jnp.float32)
    mu_ref[...] = mu
    ls_ref[...] = jnp.dot(agg, w3_ref[...], preferred_element_type=jnp.float32)
    z = jnp.concatenate([feat_ref[...], mu], axis=1)
    zs = _elu(jnp.dot(z, ws_ref[...], preferred_element_type=jnp.float32)
              * ss_ref[...] + ts_ref[...])
    zp = _elu(jnp.dot(z, wp_ref[...], preferred_element_type=jnp.float32)
              * sp_ref[...] + tp_ref[...])
    zs_ref[...] = zs
    zp_ref[...] = zp
    comb = jnp.concatenate([zs, zp], axis=1)
    dec_ref[...] = _elu(jnp.dot(comb, wd_ref[...], preferred_element_type=jnp.float32)
                        * sd_ref[...] + td_ref[...])


def _row_spec(width):
    return pl.BlockSpec((BR, width), lambda i: (i, 0))


def _p0_spec(width):
    return pl.BlockSpec((BR, width), lambda i: (i, 0))


def _p1_spec(width):
    return pl.BlockSpec((BR, width), lambda i: (i + GRID, 0))


def _full_spec(shape):
    return pl.BlockSpec(shape, lambda i: (0,) * len(shape))


def _scale_shift(b, g, be):
    s = (g * _BN).reshape(1, -1)
    t = (be + b * g * _BN).reshape(1, -1)
    return s, t


def kernel(x, edge_index, W_e1, b_e1, g_e1, be_e1, W_e2, b_e2, g_e2, be_e2,
           W1, W2, W3, W_s, b_s, g_s, be_s, W_p, b_p, g_p, be_p,
           W_d, b_d, g_d, be_d):
    s1, t1 = _scale_shift(b_e1, g_e1, be_e1)
    s2, t2 = _scale_shift(b_e2, g_e2, be_e2)
    ss, ts = _scale_shift(b_s, g_s, be_s)
    sp, tp = _scale_shift(b_p, g_p, be_p)
    sd, td = _scale_shift(b_d, g_d, be_d)

    rowi = edge_index[0]
    coli = edge_index[1]
    zeros = jnp.zeros((N, F), jnp.float32)

    feat_x = pl.pallas_call(
        _enc_body,
        grid=(GRID,),
        in_specs=[
            _row_spec(D_IN),
            _full_spec((D_IN, FH1)), _full_spec((1, FH1)), _full_spec((1, FH1)),
            _full_spec((FH1, FH2)), _full_spec((1, FH2)), _full_spec((1, FH2)),
        ],
        out_specs=_row_spec(FH2),
        out_shape=jax.ShapeDtypeStruct((N, FH2), jnp.float32),
    )(x, W_e1, s1, t1, W_e2, s2, t2)

    part1 = _spmm_sc(feat_x, rowi, coli, zeros)

    hidden1 = pl.pallas_call(
        _hidden_body,
        grid=(GRID,),
        in_specs=[_p0_spec(F), _p1_spec(F), _full_spec((FH2, GH1))],
        out_specs=_row_spec(GH1),
        out_shape=jax.ShapeDtypeStruct((N, GH1), jnp.float32),
    )(part1, part1, W1)

    part2 = _spmm_sc(hidden1, rowi, coli, zeros)

    mu, ls, zs, zp, dec = pl.pallas_call(
        _head_body,
        grid=(GRID,),
        in_specs=[
            _row_spec(FH2), _p0_spec(F), _p1_spec(F),
            _full_spec((GH1, GH2)), _full_spec((GH1, GH2)),
            _full_spec((LAT, SH)), _full_spec((1, SH)), _full_spec((1, SH)),
            _full_spec((LAT, LAT - SH)), _full_spec((1, LAT - SH)), _full_spec((1, LAT - SH)),
            _full_spec((LAT, D_IN)), _full_spec((1, D_IN)), _full_spec((1, D_IN)),
        ],
        out_specs=[
            _row_spec(GH2), _row_spec(GH2), _row_spec(SH),
            _row_spec(LAT - SH), _row_spec(D_IN),
        ],
        out_shape=[
            jax.ShapeDtypeStruct((N, GH2), jnp.float32),
            jax.ShapeDtypeStruct((N, GH2), jnp.float32),
            jax.ShapeDtypeStruct((N, SH), jnp.float32),
            jax.ShapeDtypeStruct((N, LAT - SH), jnp.float32),
            jax.ShapeDtypeStruct((N, D_IN), jnp.float32),
        ],
    )(feat_x, part2, part2, W2, W3, W_s, ss, ts, W_p, sp, tp, W_d, sd, td)

    return (mu, ls, zs, zp, dec)


# speedup vs baseline: 27.8762x; 1.0726x over previous
import functools

import jax
import jax.numpy as jnp
import numpy as np
from jax import lax
from jax.experimental import pallas as pl
from jax.experimental.pallas import tpu as pltpu
from jax.experimental.pallas import tpu_sc as plsc

N = 10000
E = 320000
D_IN = 128
FH1 = 64
FH2 = 32
GH1 = 32
GH2 = 16
SH = 16
LAT = FH2 + GH2
F = 32
_BN = 1.0 / float(np.sqrt(1.001))

NC = 2
NS = 16
NW = NC * NS
EPT = E // NW
CH = 128
NCH = EPT // CH
ETAIL = EPT - NCH * CH
NBUF = 6
PF = 4
RPT = 624
RTAIL = N - NS * RPT

BR = N
GRID = 1


_sc_mesh = plsc.VectorSubcoreMesh(
    core_axis_name="c", subcore_axis_name="s", num_cores=NC, num_subcores=NS
)


@functools.partial(
    pl.kernel,
    out_type=(jax.ShapeDtypeStruct((N, F), jnp.float32),
              jax.ShapeDtypeStruct((N, F), jnp.float32)),
    mesh=_sc_mesh,
    compiler_params=pltpu.CompilerParams(use_tc_tiling_on_sc=False),
    scratch_types=[
        pltpu.VMEM((EPT,), jnp.int32),
        pltpu.VMEM((NBUF, CH, F), jnp.float32),
        pltpu.VMEM_SHARED((N, F), jnp.float32),
        pltpu.VMEM_SHARED((N, F), jnp.float32),
        pltpu.VMEM((ETAIL,), jnp.int32),
    ]
    + [pltpu.VMEM((CH,), jnp.int32)] * NBUF
    + [pltpu.SemaphoreType.DMA] * (3 * NBUF),
)
def _spmm_sc(table, rowi, coli, zeros, out0, out1, colbuf, rows, acc,
             sp_table, rtail, *rest):
    rbuf = rest[:NBUF]
    gsems = rest[NBUF:2 * NBUF]
    ssems = rest[2 * NBUF:3 * NBUF]
    rsems = rest[3 * NBUF:]
    cid = lax.axis_index("c")
    sid = lax.axis_index("s")
    wid = sid * NC + cid

    ebase = wid * EPT
    pltpu.sync_copy(coli.at[pl.ds(ebase, EPT)], colbuf)

    pltpu.sync_copy(zeros.at[pl.ds(sid * RPT, RPT)], acc.at[pl.ds(sid * RPT, RPT)])
    pltpu.sync_copy(table.at[pl.ds(sid * RPT, RPT)], sp_table.at[pl.ds(sid * RPT, RPT)])

    @pl.when(sid == NS - 1)
    def _stage_tail():
        pltpu.sync_copy(
            zeros.at[pl.ds(NS * RPT, RTAIL)],
            acc.at[pl.ds(NS * RPT, RTAIL)],
        )
        pltpu.sync_copy(
            table.at[pl.ds(NS * RPT, RTAIL)],
            sp_table.at[pl.ds(NS * RPT, RTAIL)],
        )

    plsc.subcore_barrier()

    def _gather(i, slot):
        pltpu.async_copy(rowi.at[pl.ds(ebase + i * CH, CH)], rbuf[slot], rsems[slot])
        return pltpu.async_copy(
            sp_table.at[colbuf.at[pl.ds(i * CH, CH)]], rows.at[slot], gsems[slot]
        )

    def _gather_wait(i, slot):
        pltpu.make_async_copy(
            sp_table.at[colbuf.at[pl.ds(i * CH, CH)]], rows.at[slot], gsems[slot]
        ).wait()

    def _scatter(i, slot):
        pltpu.make_async_copy(
            rowi.at[pl.ds(ebase + i * CH, CH)], rbuf[slot], rsems[slot]
        ).wait()
        return pltpu.async_copy(rows.at[slot], acc.at[rbuf[slot]], ssems[slot], add=True)

    def _scatter_wait(slot):
        pltpu.make_async_copy(rows.at[slot], acc.at[rbuf[slot]], ssems[slot]).wait()

    for b in range(PF):
        _gather(b, b)

    @pl.loop(0, NCH // NBUF)
    def _outer(r):
        base = r * NBUF
        for b in range(NBUF):
            i = base + b
            _gather_wait(i, b)
            _scatter(i, b)
            j = i + PF
            jslot = (b + PF) % NBUF

            @pl.when(j >= NBUF)
            def _drain():
                _scatter_wait(jslot)

            @pl.when(j < NCH)
            def _prefetch():
                _gather(j, jslot)

    for b in range(NBUF - PF):
        _scatter_wait((NCH - (NBUF - PF) + b) % NBUF)

    pltpu.sync_copy(rowi.at[pl.ds(ebase + NCH * CH, ETAIL)], rtail)
    pltpu.async_copy(
        sp_table.at[colbuf.at[pl.ds(NCH * CH, ETAIL)]],
        rows.at[0, pl.ds(0, ETAIL)],
        gsems[0],
    ).wait()
    pltpu.sync_copy(rows.at[0, pl.ds(0, ETAIL)], acc.at[rtail], add=True)

    plsc.subcore_barrier()

    for c, o in ((0, out0), (1, out1)):
        @pl.when(cid == c)
        def _write(o=o):
            pltpu.sync_copy(
                acc.at[pl.ds(sid * RPT, RPT)],
                o.at[pl.ds(sid * RPT, RPT)],
            )

            @pl.when(sid == NS - 1)
            def _write_tail():
                pltpu.sync_copy(
                    acc.at[pl.ds(NS * RPT, RTAIL)],
                    o.at[pl.ds(NS * RPT, RTAIL)],
                )


def _elu(h):
    return jnp.where(h > 0.0, h, jnp.exp(jnp.minimum(h, 0.0)) - 1.0)


def _enc_body(x_ref, w1_ref, s1_ref, t1_ref, w2_ref, s2_ref, t2_ref, o_ref):
    h = jnp.dot(x_ref[...], w1_ref[...], preferred_element_type=jnp.float32)
    h = _elu(h * s1_ref[...] + t1_ref[...])
    h = jnp.dot(h, w2_ref[...], preferred_element_type=jnp.float32)
    o_ref[...] = _elu(h * s2_ref[...] + t2_ref[...])


def _hidden_body(p0_ref, p1_ref, w1_ref, o_ref):
    agg = p0_ref[...] + p1_ref[...]
    w1 = w1_ref[...]
    hs = []
    for k in range(4):
        a_k = agg[:, k * F:(k + 1) * F]
        hs.append(jnp.maximum(
            jnp.dot(a_k, w1, preferred_element_type=jnp.float32), 0.0))
    o_ref[...] = jnp.concatenate(hs, axis=1)


def _head_body(feat_ref, q0_ref, q1_ref, w2_ref, w3_ref,
               ws_ref, ss_ref, ts_ref, wp_ref, sp_ref, tp_ref,
               wd_ref, sd_ref, td_ref,
               mu_ref, ls_ref, zs_ref, zp_ref, dec_ref):
    agg = q0_ref[...] + q1_ref[...]
    feat = feat_ref[...]
    w2, w3 = w2_ref[...], w3_ref[...]
    ws, wp, wd = ws_ref[...], wp_ref[...], wd_ref[...]
    mus, lss, zss, zps, decs = [], [], [], [], []
    for k in range(4):
        a_k = agg[:, k * F:(k + 1) * F]
        f_k = feat[:, k * FH2:(k + 1) * FH2]
        mu_k = jnp.dot(a_k, w2, preferred_element_type=jnp.float32)
        mus.append(mu_k)
        lss.append(jnp.dot(a_k, w3, preferred_element_type=jnp.float32))
        z_k = jnp.concatenate([f_k, mu_k], axis=1)
        zs_k = _elu(jnp.dot(z_k, ws, preferred_element_type=jnp.float32)
                    * ss_ref[...] + ts_ref[...])
        zp_k = _elu(jnp.dot(z_k, wp, preferred_element_type=jnp.float32)
                    * sp_ref[...] + tp_ref[...])
        zss.append(zs_k)
        zps.append(zp_k)
        comb_k = jnp.concatenate([zs_k, zp_k], axis=1)
        decs.append(_elu(jnp.dot(comb_k, wd, preferred_element_type=jnp.float32)
                         * sd_ref[...] + td_ref[...]))
    mu_ref[...] = jnp.concatenate(mus, axis=1)
    ls_ref[...] = jnp.concatenate(lss, axis=1)
    zs_ref[...] = jnp.concatenate(zss, axis=1)
    zp_ref[...] = jnp.concatenate(zps, axis=1)
    dec_ref[...] = jnp.concatenate(decs, axis=1)


def _row_spec(width):
    return pl.BlockSpec((N, width), lambda i: (0, 0))


def _pk_spec():
    return pl.BlockSpec((N // 4, 128), lambda i: (0, 0))


def _full_spec(shape):
    return pl.BlockSpec(shape, lambda i: (0,) * len(shape))


def _scale_shift(b, g, be):
    s = (g * _BN).reshape(1, -1)
    t = (be + b * g * _BN).reshape(1, -1)
    return s, t


def kernel(x, edge_index, W_e1, b_e1, g_e1, be_e1, W_e2, b_e2, g_e2, be_e2,
           W1, W2, W3, W_s, b_s, g_s, be_s, W_p, b_p, g_p, be_p,
           W_d, b_d, g_d, be_d):
    s1, t1 = _scale_shift(b_e1, g_e1, be_e1)
    s2, t2 = _scale_shift(b_e2, g_e2, be_e2)
    ss, ts = _scale_shift(b_s, g_s, be_s)
    sp, tp = _scale_shift(b_p, g_p, be_p)
    sd, td = _scale_shift(b_d, g_d, be_d)

    rowi = edge_index[0]
    coli = edge_index[1]
    zeros = jnp.zeros((N, F), jnp.float32)

    feat_x = pl.pallas_call(
        _enc_body,
        grid=(GRID,),
        in_specs=[
            _row_spec(D_IN),
            _full_spec((D_IN, FH1)), _full_spec((1, FH1)), _full_spec((1, FH1)),
            _full_spec((FH1, FH2)), _full_spec((1, FH2)), _full_spec((1, FH2)),
        ],
        out_specs=_row_spec(FH2),
        out_shape=jax.ShapeDtypeStruct((N, FH2), jnp.float32),
    )(x, W_e1, s1, t1, W_e2, s2, t2)

    p10, p11 = _spmm_sc(feat_x, rowi, coli, zeros)
    feat_pk = feat_x.reshape(N // 4, 128)

    hidden_pk = pl.pallas_call(
        _hidden_body,
        grid=(GRID,),
        in_specs=[_pk_spec(), _pk_spec(), _full_spec((FH2, GH1))],
        out_specs=_pk_spec(),
        out_shape=jax.ShapeDtypeStruct((N // 4, 128), jnp.float32),
    )(p10.reshape(N // 4, 128), p11.reshape(N // 4, 128), W1)

    p20, p21 = _spmm_sc(hidden_pk.reshape(N, GH1), rowi, coli, zeros)

    mu_pk, ls_pk, zs_pk, zp_pk, dec_pk = pl.pallas_call(
        _head_body,
        grid=(GRID,),
        in_specs=[
            _pk_spec(), _pk_spec(), _pk_spec(),
            _full_spec((GH1, GH2)), _full_spec((GH1, GH2)),
            _full_spec((LAT, SH)), _full_spec((1, SH)), _full_spec((1, SH)),
            _full_spec((LAT, LAT - SH)), _full_spec((1, LAT - SH)), _full_spec((1, LAT - SH)),
            _full_spec((LAT, D_IN)), _full_spec((1, D_IN)), _full_spec((1, D_IN)),
        ],
        out_specs=[
            _full_spec((N // 4, 4 * GH2)), _full_spec((N // 4, 4 * GH2)),
            _full_spec((N // 4, 4 * SH)), _full_spec((N // 4, 4 * (LAT - SH))),
            _full_spec((N // 4, 4 * D_IN)),
        ],
        out_shape=[
            jax.ShapeDtypeStruct((N // 4, 4 * GH2), jnp.float32),
            jax.ShapeDtypeStruct((N // 4, 4 * GH2), jnp.float32),
            jax.ShapeDtypeStruct((N // 4, 4 * SH), jnp.float32),
            jax.ShapeDtypeStruct((N // 4, 4 * (LAT - SH)), jnp.float32),
            jax.ShapeDtypeStruct((N // 4, 4 * D_IN), jnp.float32),
        ],
    )(feat_pk, p20.reshape(N // 4, 128), p21.reshape(N // 4, 128),
      W2, W3, W_s, ss, ts, W_p, sp, tp, W_d, sd, td)

    return (mu_pk.reshape(N, GH2), ls_pk.reshape(N, GH2), zs_pk.reshape(N, SH),
            zp_pk.reshape(N, LAT - SH), dec_pk.reshape(N, D_IN))
